# baseline (speedup 1.0000x reference)
import functools

import jax
import jax.numpy as jnp
from jax.experimental import pallas as pl
from jax.experimental.pallas import tpu as pltpu

T, D, E, FF = 2048, 2048, 8, 1024
BT = 256
BF = 512
NF = FF // BF


def _router_body(x_ref, rw_ref, ew_ref):
    x = x_ref[...]
    rw = rw_ref[...]
    logits = jax.lax.dot_general(
        x, rw, (((1,), (1,)), ((), ())), preferred_element_type=jnp.float32)
    m = jnp.max(logits, axis=-1, keepdims=True)
    p = jnp.exp(logits - m)
    p = p / jnp.sum(p, axis=-1, keepdims=True)
    ii = jax.lax.broadcasted_iota(jnp.int32, p.shape, 1)
    m1 = jnp.max(p, axis=-1, keepdims=True)
    i1 = jnp.min(jnp.where(p >= m1, ii, E), axis=-1, keepdims=True)
    p2 = jnp.where(ii == i1, -jnp.inf, p)
    m2 = jnp.max(p2, axis=-1, keepdims=True)
    i2 = jnp.min(jnp.where(p2 >= m2, ii, E), axis=-1, keepdims=True)
    s = m1 + m2
    ew = jnp.where(ii == i1, m1 / s, 0.0) + jnp.where(ii == i2, m2 / s, 0.0)
    ew_ref[...] = jnp.concatenate(
        [ew, jnp.zeros((ew.shape[0], 128 - E), jnp.float32)], axis=1)


def _router(x, rw):
    return pl.pallas_call(
        _router_body,
        grid=(T // BT,),
        in_specs=[
            pl.BlockSpec((BT, D), lambda i: (i, 0)),
            pl.BlockSpec((E, D), lambda i: (0, 0)),
        ],
        out_specs=pl.BlockSpec((BT, 128), lambda i: (i, 0)),
        out_shape=jax.ShapeDtypeStruct((T, 128), jnp.float32),
    )(x, rw)


def _gateup_body(x_ref, wg_ref, wu_ref, ew_ref, act_ref):
    e = pl.program_id(0)
    x = x_ref[...]
    wg = wg_ref[0]
    wu = wu_ref[0]
    g = jax.lax.dot_general(
        x, wg, (((1,), (1,)), ((), ())), preferred_element_type=jnp.float32)
    u = jax.lax.dot_general(
        x, wu, (((1,), (1,)), ((), ())), preferred_element_type=jnp.float32)
    a = (g * jax.nn.sigmoid(g)) * u
    ewb = ew_ref[...]
    ii = jax.lax.broadcasted_iota(jnp.int32, ewb.shape, 1)
    w = jnp.sum(jnp.where(ii == e, ewb, 0.0), axis=1)
    act_ref[0] = a * w[:, None]


def _gateup(x, w13, ewt):
    return pl.pallas_call(
        _gateup_body,
        grid=(E, NF, T // BT),
        in_specs=[
            pl.BlockSpec((BT, D), lambda e, f, i: (i, 0)),
            pl.BlockSpec((1, BF, D), lambda e, f, i: (e, f, 0)),
            pl.BlockSpec((1, BF, D), lambda e, f, i: (e, f + NF, 0)),
            pl.BlockSpec((BT, 128), lambda e, f, i: (i, 0)),
        ],
        out_specs=pl.BlockSpec((1, BT, BF), lambda e, f, i: (e, i, f)),
        out_shape=jax.ShapeDtypeStruct((E, T, FF), jnp.float32),
    )(x, w13, w13, ewt)


def _down_body(act_ref, w2_ref, out_ref):
    e = pl.program_id(1)
    a = act_ref[0]
    w2 = w2_ref[0]
    part = jax.lax.dot_general(
        a, w2, (((1,), (1,)), ((), ())), preferred_element_type=jnp.float32)

    @pl.when(e == 0)
    def _init():
        out_ref[...] = part

    @pl.when(e != 0)
    def _acc():
        out_ref[...] = out_ref[...] + part


def _down(act, w2):
    return pl.pallas_call(
        _down_body,
        grid=(T // BT, E),
        in_specs=[
            pl.BlockSpec((1, BT, FF), lambda i, e: (e, i, 0)),
            pl.BlockSpec((1, D, FF), lambda i, e: (e, 0, 0)),
        ],
        out_specs=pl.BlockSpec((BT, D), lambda i, e: (i, 0)),
        out_shape=jax.ShapeDtypeStruct((T, D), jnp.float32),
    )(act, w2)


def kernel(hidden_states, router_weight, w13, w2):
    ew = _router(hidden_states, router_weight)
    act = _gateup(hidden_states, w13, ew)
    return _down(act, w2)
